# baseline (speedup 1.0000x reference)
import functools

import jax
import jax.numpy as jnp
from jax import lax
from jax.experimental import pallas as pl
from jax.experimental.pallas import tpu as pltpu
from jax.experimental.pallas import tpu_sc as plsc

NN = 10000
EE = 320000
DIN = 128
NH = 8
CH = 16
D2 = 64
NEG = -1e30
BN = 1000
CHK = 100
NW = 32
RPT = 624
TAIL = NN - 16 * RPT
ZC = 96
W1C = DIN + 16
W2C = D2 + 16


def _tc1_body(x_r, w1_r, pa_r, pd_r, pads_r, hc_r, ad_r, ms_r, md_r):
    i = pl.program_id(0)
    h = jnp.dot(x_r[...], w1_r[...], preferred_element_type=jnp.float32)
    a_s = jnp.dot(h, pa_r[...], preferred_element_type=jnp.float32) + pads_r[...]
    a_d = jnp.dot(h, pd_r[...], preferred_element_type=jnp.float32)
    hc_r[...] = jnp.concatenate([h, a_s], axis=1)
    ad_r[...] = a_d
    cs = jnp.max(a_s, axis=0, keepdims=True)
    cd = jnp.max(a_d, axis=0, keepdims=True)

    @pl.when(i == 0)
    def _():
        ms_r[...] = cs
        md_r[...] = cd

    @pl.when(i != 0)
    def _():
        ms_r[...] = jnp.maximum(ms_r[...], cs)
        md_r[...] = jnp.maximum(md_r[...], cd)


def _tc1(x, w1, pa, pd, pads):
    return pl.pallas_call(
        _tc1_body,
        grid=(NN // BN,),
        in_specs=[
            pl.BlockSpec((BN, DIN), lambda i: (i, 0)),
            pl.BlockSpec((DIN, DIN), lambda i: (0, 0)),
            pl.BlockSpec((DIN, 16), lambda i: (0, 0)),
            pl.BlockSpec((DIN, 16), lambda i: (0, 0)),
            pl.BlockSpec((1, 16), lambda i: (0, 0)),
        ],
        out_specs=[
            pl.BlockSpec((BN, W1C), lambda i: (i, 0)),
            pl.BlockSpec((BN, 16), lambda i: (i, 0)),
            pl.BlockSpec((1, 16), lambda i: (0, 0)),
            pl.BlockSpec((1, 16), lambda i: (0, 0)),
        ],
        out_shape=[
            jax.ShapeDtypeStruct((NN, W1C), jnp.float32),
            jax.ShapeDtypeStruct((NN, 16), jnp.float32),
            jax.ShapeDtypeStruct((1, 16), jnp.float32),
            jax.ShapeDtypeStruct((1, 16), jnp.float32),
        ],
    )(x, w1, pa, pd, pads)


def _sc_edge_body(nfeat, chp, kb, nchk, nb,
                  src2_h, dst2_h, hc_h, ad_h, m_h, acc_hbm,
                  sidx_a, didx_a, ad0, h0, ad1, h1, m_b,
                  acc_sp, g0, g1, a0, a1):
    nv = nfeat // 16
    w = nfeat + 16
    cb = nb // 5
    cid = lax.axis_index("c")
    sid = lax.axis_index("s")
    wid = cid * 16 + sid

    def fire_gather(lb, ad_b, h_b, gs):
        for j in range(nchk):
            pltpu.async_copy(hc_h.at[sidx_a.at[lb + j]],
                             h_b.at[pl.ds(j * CHK, CHK)], gs)
            pltpu.async_copy(ad_h.at[didx_a.at[lb + j]],
                             ad_b.at[pl.ds(j * CHK, CHK)], gs)

    def drain_gather(ad_b, h_b, gs):
        pltpu.make_async_copy(hc_h.at[pl.ds(0, kb)], h_b, gs).wait()
        pltpu.make_async_copy(ad_h.at[pl.ds(0, kb)], ad_b, gs).wait()

    def fire_scatter(lb, h_b, sa):
        for j in range(nchk):
            pltpu.async_copy(h_b.at[pl.ds(j * CHK, CHK)],
                             acc_sp.at[didx_a.at[lb + j]], sa, add=True)

    def drain_scatter(h_b, sa):
        pltpu.make_async_copy(hc_h.at[pl.ds(0, kb)], h_b, sa).wait()

    def compute(ad_b, h_b, mv):
        dn = lax.GatherDimensionNumbers(
            offset_dims=(), collapsed_slice_dims=(0,), start_index_map=(0,))

        def edge_body(k, m):
            e = h_b[k, pl.ds(nfeat, 16)] + ad_b[k]
            e = jnp.maximum(e, 0.2 * e)
            ex = jnp.exp(e - m)
            h_b[k, pl.ds(nfeat, 16)] = ex
            for j in range(nv):
                idx = jnp.full((16, 1), j * 16 // chp, jnp.int32)
                s = lax.gather(ex, idx, dn, slice_sizes=(1,),
                               mode=lax.GatherScatterMode.PROMISE_IN_BOUNDS)
                h_b[k, pl.ds(j * 16, 16)] = h_b[k, pl.ds(j * 16, 16)] * s
            return m

        lax.fori_loop(0, kb, edge_body, mv, unroll=2)

    def z_body(k, c):
        for j in range(w // 16):
            h0[k, pl.ds(j * 16, 16)] = jnp.zeros((16,), jnp.float32)
        return c

    lax.fori_loop(0, kb, z_body, 0)
    r0 = sid * RPT

    def zi_body(i, c):
        pltpu.sync_copy(h0.at[pl.ds(0, ZC)], acc_sp.at[pl.ds(r0 + i * ZC, ZC)])
        return c

    lax.fori_loop(0, RPT // ZC, zi_body, 0)
    zr = RPT % ZC
    if zr:
        pltpu.sync_copy(h0.at[pl.ds(0, zr)], acc_sp.at[pl.ds(r0 + RPT - zr, zr)])

    @pl.when(sid == 15)
    def _():
        pltpu.sync_copy(h0.at[pl.ds(0, TAIL)], acc_sp.at[pl.ds(16 * RPT, TAIL)])

    pltpu.sync_copy(m_h, m_b)
    plsc.subcore_barrier()
    mv0 = m_b[...]
    rc = cb * nchk

    def chunk_body(c, mv):
        @pl.when(c > 0)
        def _():
            drain_scatter(h1, a1)

        row0 = (wid * nb + c * cb) * nchk
        pltpu.sync_copy(src2_h.at[pl.ds(row0, rc)], sidx_a)
        pltpu.sync_copy(dst2_h.at[pl.ds(row0, rc)], didx_a)
        fire_gather(0, ad0, h0, g0)

        def inner(i, mv2):
            lb0 = 2 * i * nchk
            lb1 = lb0 + nchk
            drain_gather(ad0, h0, g0)

            @pl.when(i > 0)
            def _():
                drain_scatter(h1, a1)

            fire_gather(lb1, ad1, h1, g1)
            compute(ad0, h0, mv2)
            fire_scatter(lb0, h0, a0)
            drain_gather(ad1, h1, g1)
            compute(ad1, h1, mv2)
            drain_scatter(h0, a0)

            @pl.when(i < cb // 2 - 1)
            def _():
                fire_gather(lb1 + nchk, ad0, h0, g0)

            fire_scatter(lb1, h1, a1)
            return mv2

        lax.fori_loop(0, cb // 2, inner, mv)
        return mv

    lax.fori_loop(0, 5, chunk_body, mv0)
    drain_scatter(h1, a1)
    plsc.subcore_barrier()
    pltpu.sync_copy(acc_sp.at[pl.ds(r0, RPT)], acc_hbm.at[cid, pl.ds(r0, RPT)])

    @pl.when(sid == 15)
    def _():
        pltpu.sync_copy(acc_sp.at[pl.ds(16 * RPT, TAIL)],
                        acc_hbm.at[cid, pl.ds(16 * RPT, TAIL)])


def _make_sc_edge(nfeat, chp, kb, nchk, nb):
    rc = (nb // 5) * nchk
    w = nfeat + 16
    mesh = plsc.VectorSubcoreMesh(core_axis_name="c", subcore_axis_name="s")
    return pl.kernel(
        functools.partial(_sc_edge_body, nfeat, chp, kb, nchk, nb),
        out_type=jax.ShapeDtypeStruct((2, NN, w), jnp.float32),
        mesh=mesh,
        scratch_types=[
            pltpu.VMEM((rc, CHK), jnp.int32),
            pltpu.VMEM((rc, CHK), jnp.int32),
            pltpu.VMEM((kb, 16), jnp.float32),
            pltpu.VMEM((kb, w), jnp.float32),
            pltpu.VMEM((kb, 16), jnp.float32),
            pltpu.VMEM((kb, w), jnp.float32),
            pltpu.VMEM((16,), jnp.float32),
            pltpu.VMEM_SHARED((NN, w), jnp.float32),
            pltpu.SemaphoreType.DMA,
            pltpu.SemaphoreType.DMA,
            pltpu.SemaphoreType.DMA,
            pltpu.SemaphoreType.DMA,
        ],
        compiler_params=pltpu.CompilerParams(use_tc_tiling_on_sc=False),
    )


def _tc2_body(accp_r, hc_r, ad_r, m1_r, b1_r, pt_r, w2_r,
              aa2_r, ad2_r, pad2_r, hc2_r, ad2o_r, ms2_r, md2_r):
    i = pl.program_id(0)
    hc = hc_r[...]
    h1 = hc[:, 0:DIN]
    a_s1 = hc[:, DIN:W1C]
    le = a_s1 + ad_r[...]
    le = jnp.maximum(le, 0.2 * le)
    lex = jnp.exp(le - m1_r[...])
    accp = accp_r[...]
    den16 = accp[0, :, DIN:W1C] + accp[1, :, DIN:W1C] + lex
    acc = accp[0, :, 0:DIN] + accp[1, :, 0:DIN] + h1 * jnp.dot(
        lex, pt_r[...], preferred_element_type=jnp.float32)
    den = jnp.dot(den16, pt_r[...], preferred_element_type=jnp.float32)
    out1 = acc / (den + 1e-16) + b1_r[...]
    x2 = jnp.where(out1 > 0, out1, jnp.exp(jnp.minimum(out1, 0.0)) - 1.0)
    h2 = jnp.dot(x2, w2_r[...], preferred_element_type=jnp.float32)
    a_s = jnp.dot(h2, aa2_r[...], preferred_element_type=jnp.float32) + pad2_r[...]
    a_d = jnp.dot(h2, ad2_r[...], preferred_element_type=jnp.float32)
    hc2_r[...] = jnp.concatenate([h2, a_s], axis=1)
    ad2o_r[...] = a_d
    cs = jnp.max(a_s, axis=0, keepdims=True)
    cd = jnp.max(a_d, axis=0, keepdims=True)

    @pl.when(i == 0)
    def _():
        ms2_r[...] = cs
        md2_r[...] = cd

    @pl.when(i != 0)
    def _():
        ms2_r[...] = jnp.maximum(ms2_r[...], cs)
        md2_r[...] = jnp.maximum(md2_r[...], cd)


def _tc2(accp, hc1, ad1p, m1v, b1, pt, w2, aa2, ad2, pad2):
    return pl.pallas_call(
        _tc2_body,
        grid=(NN // BN,),
        in_specs=[
            pl.BlockSpec((2, BN, W1C), lambda i: (0, i, 0)),
            pl.BlockSpec((BN, W1C), lambda i: (i, 0)),
            pl.BlockSpec((BN, 16), lambda i: (i, 0)),
            pl.BlockSpec((1, 16), lambda i: (0, 0)),
            pl.BlockSpec((1, DIN), lambda i: (0, 0)),
            pl.BlockSpec((16, DIN), lambda i: (0, 0)),
            pl.BlockSpec((DIN, D2), lambda i: (0, 0)),
            pl.BlockSpec((D2, 16), lambda i: (0, 0)),
            pl.BlockSpec((D2, 16), lambda i: (0, 0)),
            pl.BlockSpec((1, 16), lambda i: (0, 0)),
        ],
        out_specs=[
            pl.BlockSpec((BN, W2C), lambda i: (i, 0)),
            pl.BlockSpec((BN, 16), lambda i: (i, 0)),
            pl.BlockSpec((1, 16), lambda i: (0, 0)),
            pl.BlockSpec((1, 16), lambda i: (0, 0)),
        ],
        out_shape=[
            jax.ShapeDtypeStruct((NN, W2C), jnp.float32),
            jax.ShapeDtypeStruct((NN, 16), jnp.float32),
            jax.ShapeDtypeStruct((1, 16), jnp.float32),
            jax.ShapeDtypeStruct((1, 16), jnp.float32),
        ],
    )(accp, hc1, ad1p, m1v, b1, pt, w2, aa2, ad2, pad2)


def _tc3_body(accp_r, hc2_r, ad_r, m2_r, b2_r, pt2_r, out_r):
    hc2 = hc2_r[...]
    h2 = hc2[:, 0:D2]
    a_s2 = hc2[:, D2:W2C]
    le = a_s2 + ad_r[...]
    le = jnp.maximum(le, 0.2 * le)
    lex = jnp.exp(le - m2_r[...])
    accp = accp_r[...]
    den16 = accp[0, :, D2:W2C] + accp[1, :, D2:W2C] + lex
    den = jnp.dot(den16, pt2_r[...], preferred_element_type=jnp.float32)
    esp = jnp.dot(lex, pt2_r[...], preferred_element_type=jnp.float32)
    acc = accp[0, :, 0:D2] + accp[1, :, 0:D2] + h2 * esp
    out2 = acc / (den + 1e-16) + b2_r[...]
    mx = jnp.max(out2, axis=1, keepdims=True)
    sh = out2 - mx
    out_r[...] = sh - jnp.log(jnp.sum(jnp.exp(sh), axis=1, keepdims=True))


def _tc3(accp, hc2, ad2p, m2v, b2, pt2):
    return pl.pallas_call(
        _tc3_body,
        grid=(NN // BN,),
        in_specs=[
            pl.BlockSpec((2, BN, W2C), lambda i: (0, i, 0)),
            pl.BlockSpec((BN, W2C), lambda i: (i, 0)),
            pl.BlockSpec((BN, 16), lambda i: (i, 0)),
            pl.BlockSpec((1, 16), lambda i: (0, 0)),
            pl.BlockSpec((1, D2), lambda i: (0, 0)),
            pl.BlockSpec((16, D2), lambda i: (0, 0)),
        ],
        out_specs=pl.BlockSpec((BN, D2), lambda i: (i, 0)),
        out_shape=jax.ShapeDtypeStruct((NN, D2), jnp.float32),
    )(accp, hc2, ad2p, m2v, b2, pt2)


def kernel(x, edge_index, W1, att_src1, att_dst1, b1, W2, att_src2,
           att_dst2, b2):
    lane = jnp.arange(16)
    hof = jax.nn.one_hot(jnp.arange(DIN) // CH, 16, dtype=jnp.float32)
    pa = hof * att_src1.reshape(DIN)[:, None]
    pd = hof * att_dst1.reshape(DIN)[:, None]
    pads = jnp.where(lane < NH, 0.0, NEG).reshape(1, 16).astype(jnp.float32)
    pt = hof.T

    hc1, ad1p, ms1, md1 = _tc1(x, W1, pa, pd, pads)

    m1 = ms1 + md1
    m1 = jnp.maximum(m1, 0.2 * m1)
    m1v = jnp.where(lane < NH, m1, 1e30).astype(jnp.float32)

    src2 = edge_index[0].reshape(EE // CHK, CHK).astype(jnp.int32)
    dst2 = edge_index[1].reshape(EE // CHK, CHK).astype(jnp.int32)

    sc1 = _make_sc_edge(DIN, CH, 100, 1, EE // (NW * 100))
    accp = sc1(src2, dst2, hc1, ad1p, m1v.reshape(16))

    aa2 = jnp.where(lane[None, :] == 0, att_src2.reshape(D2)[:, None], 0.0)
    ad2 = jnp.where(lane[None, :] == 0, att_dst2.reshape(D2)[:, None], 0.0)
    pad2 = jnp.where(lane == 0, 0.0, NEG).reshape(1, 16).astype(jnp.float32)

    hc2, ad2p, ms2, md2 = _tc2(
        accp, hc1, ad1p, m1v.reshape(1, 16), b1.reshape(1, DIN),
        pt, W2, aa2, ad2, pad2)

    m2 = ms2 + md2
    m2 = jnp.maximum(m2, 0.2 * m2)
    m2v = jnp.where(lane == 0, m2, 1e30).astype(jnp.float32)

    sc2 = _make_sc_edge(D2, D2, 200, 2, EE // (NW * 200))
    acc2p = sc2(src2, dst2, hc2, ad2p, m2v.reshape(16))

    pt2 = jnp.where(lane[:, None] == 0, 1.0, 0.0) * jnp.ones((16, D2))
    return _tc3(acc2p, hc2, ad2p, m2v.reshape(1, 16),
                b2.reshape(1, D2), pt2.astype(jnp.float32))


# speedup vs baseline: 1.0001x; 1.0001x over previous
import functools

import jax
import jax.numpy as jnp
from jax import lax
from jax.experimental import pallas as pl
from jax.experimental.pallas import tpu as pltpu
from jax.experimental.pallas import tpu_sc as plsc

NN = 10000
EE = 320000
DIN = 128
NH = 8
CH = 16
D2 = 64
NEG = -1e30
BN = 1000
CHK = 100
NW = 32
RPT = 624
TAIL = NN - 16 * RPT
ZC = 96
W1C = DIN + 16
W2C = D2 + 16


def _tc1_body(x_r, w1_r, pa_r, pd_r, pads_r, hc_r, ad_r, ms_r, md_r):
    i = pl.program_id(0)
    h = jnp.dot(x_r[...], w1_r[...], preferred_element_type=jnp.float32)
    a_s = jnp.dot(h, pa_r[...], preferred_element_type=jnp.float32) + pads_r[...]
    a_d = jnp.dot(h, pd_r[...], preferred_element_type=jnp.float32)
    hc_r[...] = jnp.concatenate([h, a_s], axis=1)
    ad_r[...] = a_d
    cs = jnp.max(a_s, axis=0, keepdims=True)
    cd = jnp.max(a_d, axis=0, keepdims=True)

    @pl.when(i == 0)
    def _():
        ms_r[...] = cs
        md_r[...] = cd

    @pl.when(i != 0)
    def _():
        ms_r[...] = jnp.maximum(ms_r[...], cs)
        md_r[...] = jnp.maximum(md_r[...], cd)


def _tc1(x, w1, pa, pd, pads):
    return pl.pallas_call(
        _tc1_body,
        grid=(NN // BN,),
        in_specs=[
            pl.BlockSpec((BN, DIN), lambda i: (i, 0)),
            pl.BlockSpec((DIN, DIN), lambda i: (0, 0)),
            pl.BlockSpec((DIN, 16), lambda i: (0, 0)),
            pl.BlockSpec((DIN, 16), lambda i: (0, 0)),
            pl.BlockSpec((1, 16), lambda i: (0, 0)),
        ],
        out_specs=[
            pl.BlockSpec((BN, W1C), lambda i: (i, 0)),
            pl.BlockSpec((BN, 16), lambda i: (i, 0)),
            pl.BlockSpec((1, 16), lambda i: (0, 0)),
            pl.BlockSpec((1, 16), lambda i: (0, 0)),
        ],
        out_shape=[
            jax.ShapeDtypeStruct((NN, W1C), jnp.float32),
            jax.ShapeDtypeStruct((NN, 16), jnp.float32),
            jax.ShapeDtypeStruct((1, 16), jnp.float32),
            jax.ShapeDtypeStruct((1, 16), jnp.float32),
        ],
    )(x, w1, pa, pd, pads)


def _sc_edge_body(nfeat, chp, kb, nchk, nb,
                  src2_h, dst2_h, hc_h, ad_h, m_h, acc_hbm,
                  sidx_a, didx_a, ad0, h0, ad1, h1, m_b,
                  acc_sp, g0, g1, a0, a1):
    nv = nfeat // 16
    w = nfeat + 16
    cb = nb // 5
    cid = lax.axis_index("c")
    sid = lax.axis_index("s")
    wid = cid * 16 + sid

    def fire_gather(lb, ad_b, h_b, gs):
        for j in range(nchk):
            pltpu.async_copy(hc_h.at[sidx_a.at[lb + j]],
                             h_b.at[pl.ds(j * CHK, CHK)], gs)
            pltpu.async_copy(ad_h.at[didx_a.at[lb + j]],
                             ad_b.at[pl.ds(j * CHK, CHK)], gs)

    def drain_gather(ad_b, h_b, gs):
        pltpu.make_async_copy(hc_h.at[pl.ds(0, kb)], h_b, gs).wait()
        pltpu.make_async_copy(ad_h.at[pl.ds(0, kb)], ad_b, gs).wait()

    def fire_scatter(lb, h_b, sa):
        for j in range(nchk):
            pltpu.async_copy(h_b.at[pl.ds(j * CHK, CHK)],
                             acc_sp.at[didx_a.at[lb + j]], sa, add=True)

    def drain_scatter(h_b, sa):
        pltpu.make_async_copy(hc_h.at[pl.ds(0, kb)], h_b, sa).wait()

    def compute(ad_b, h_b, mv):
        dn = lax.GatherDimensionNumbers(
            offset_dims=(), collapsed_slice_dims=(0,), start_index_map=(0,))

        def edge_body(k, m):
            e = h_b[k, pl.ds(nfeat, 16)] + ad_b[k]
            e = jnp.maximum(e, 0.2 * e)
            ex = jnp.exp(e - m)
            h_b[k, pl.ds(nfeat, 16)] = ex
            for j in range(nv):
                idx = jnp.full((16, 1), j * 16 // chp, jnp.int32)
                s = lax.gather(ex, idx, dn, slice_sizes=(1,),
                               mode=lax.GatherScatterMode.PROMISE_IN_BOUNDS)
                h_b[k, pl.ds(j * 16, 16)] = h_b[k, pl.ds(j * 16, 16)] * s
            return m

        lax.fori_loop(0, kb, edge_body, mv)

    def z_body(k, c):
        for j in range(w // 16):
            h0[k, pl.ds(j * 16, 16)] = jnp.zeros((16,), jnp.float32)
        return c

    lax.fori_loop(0, kb, z_body, 0)
    r0 = sid * RPT

    def zi_body(i, c):
        pltpu.sync_copy(h0.at[pl.ds(0, ZC)], acc_sp.at[pl.ds(r0 + i * ZC, ZC)])
        return c

    lax.fori_loop(0, RPT // ZC, zi_body, 0)
    zr = RPT % ZC
    if zr:
        pltpu.sync_copy(h0.at[pl.ds(0, zr)], acc_sp.at[pl.ds(r0 + RPT - zr, zr)])

    @pl.when(sid == 15)
    def _():
        pltpu.sync_copy(h0.at[pl.ds(0, TAIL)], acc_sp.at[pl.ds(16 * RPT, TAIL)])

    pltpu.sync_copy(m_h, m_b)
    plsc.subcore_barrier()
    mv0 = m_b[...]
    rc = cb * nchk

    def chunk_body(c, mv):
        @pl.when(c > 0)
        def _():
            drain_scatter(h1, a1)

        row0 = (wid * nb + c * cb) * nchk
        pltpu.sync_copy(src2_h.at[pl.ds(row0, rc)], sidx_a)
        pltpu.sync_copy(dst2_h.at[pl.ds(row0, rc)], didx_a)
        fire_gather(0, ad0, h0, g0)

        def inner(i, mv2):
            lb0 = 2 * i * nchk
            lb1 = lb0 + nchk
            drain_gather(ad0, h0, g0)

            @pl.when(i > 0)
            def _():
                drain_scatter(h1, a1)

            fire_gather(lb1, ad1, h1, g1)
            compute(ad0, h0, mv2)
            fire_scatter(lb0, h0, a0)
            drain_gather(ad1, h1, g1)
            compute(ad1, h1, mv2)
            drain_scatter(h0, a0)

            @pl.when(i < cb // 2 - 1)
            def _():
                fire_gather(lb1 + nchk, ad0, h0, g0)

            fire_scatter(lb1, h1, a1)
            return mv2

        lax.fori_loop(0, cb // 2, inner, mv)
        return mv

    lax.fori_loop(0, 5, chunk_body, mv0)
    drain_scatter(h1, a1)
    plsc.subcore_barrier()
    pltpu.sync_copy(acc_sp.at[pl.ds(r0, RPT)], acc_hbm.at[cid, pl.ds(r0, RPT)])

    @pl.when(sid == 15)
    def _():
        pltpu.sync_copy(acc_sp.at[pl.ds(16 * RPT, TAIL)],
                        acc_hbm.at[cid, pl.ds(16 * RPT, TAIL)])


def _make_sc_edge(nfeat, chp, kb, nchk, nb):
    rc = (nb // 5) * nchk
    w = nfeat + 16
    mesh = plsc.VectorSubcoreMesh(core_axis_name="c", subcore_axis_name="s")
    return pl.kernel(
        functools.partial(_sc_edge_body, nfeat, chp, kb, nchk, nb),
        out_type=jax.ShapeDtypeStruct((2, NN, w), jnp.float32),
        mesh=mesh,
        scratch_types=[
            pltpu.VMEM((rc, CHK), jnp.int32),
            pltpu.VMEM((rc, CHK), jnp.int32),
            pltpu.VMEM((kb, 16), jnp.float32),
            pltpu.VMEM((kb, w), jnp.float32),
            pltpu.VMEM((kb, 16), jnp.float32),
            pltpu.VMEM((kb, w), jnp.float32),
            pltpu.VMEM((16,), jnp.float32),
            pltpu.VMEM_SHARED((NN, w), jnp.float32),
            pltpu.SemaphoreType.DMA,
            pltpu.SemaphoreType.DMA,
            pltpu.SemaphoreType.DMA,
            pltpu.SemaphoreType.DMA,
        ],
        compiler_params=pltpu.CompilerParams(use_tc_tiling_on_sc=False),
    )


def _tc2_body(accp_r, hc_r, ad_r, m1_r, b1_r, pt_r, w2_r,
              aa2_r, ad2_r, pad2_r, hc2_r, ad2o_r, ms2_r, md2_r):
    i = pl.program_id(0)
    hc = hc_r[...]
    h1 = hc[:, 0:DIN]
    a_s1 = hc[:, DIN:W1C]
    le = a_s1 + ad_r[...]
    le = jnp.maximum(le, 0.2 * le)
    lex = jnp.exp(le - m1_r[...])
    accp = accp_r[...]
    den16 = accp[0, :, DIN:W1C] + accp[1, :, DIN:W1C] + lex
    acc = accp[0, :, 0:DIN] + accp[1, :, 0:DIN] + h1 * jnp.dot(
        lex, pt_r[...], preferred_element_type=jnp.float32)
    den = jnp.dot(den16, pt_r[...], preferred_element_type=jnp.float32)
    out1 = acc / (den + 1e-16) + b1_r[...]
    x2 = jnp.where(out1 > 0, out1, jnp.exp(jnp.minimum(out1, 0.0)) - 1.0)
    h2 = jnp.dot(x2, w2_r[...], preferred_element_type=jnp.float32)
    a_s = jnp.dot(h2, aa2_r[...], preferred_element_type=jnp.float32) + pad2_r[...]
    a_d = jnp.dot(h2, ad2_r[...], preferred_element_type=jnp.float32)
    hc2_r[...] = jnp.concatenate([h2, a_s], axis=1)
    ad2o_r[...] = a_d
    cs = jnp.max(a_s, axis=0, keepdims=True)
    cd = jnp.max(a_d, axis=0, keepdims=True)

    @pl.when(i == 0)
    def _():
        ms2_r[...] = cs
        md2_r[...] = cd

    @pl.when(i != 0)
    def _():
        ms2_r[...] = jnp.maximum(ms2_r[...], cs)
        md2_r[...] = jnp.maximum(md2_r[...], cd)


def _tc2(accp, hc1, ad1p, m1v, b1, pt, w2, aa2, ad2, pad2):
    return pl.pallas_call(
        _tc2_body,
        grid=(NN // BN,),
        in_specs=[
            pl.BlockSpec((2, BN, W1C), lambda i: (0, i, 0)),
            pl.BlockSpec((BN, W1C), lambda i: (i, 0)),
            pl.BlockSpec((BN, 16), lambda i: (i, 0)),
            pl.BlockSpec((1, 16), lambda i: (0, 0)),
            pl.BlockSpec((1, DIN), lambda i: (0, 0)),
            pl.BlockSpec((16, DIN), lambda i: (0, 0)),
            pl.BlockSpec((DIN, D2), lambda i: (0, 0)),
            pl.BlockSpec((D2, 16), lambda i: (0, 0)),
            pl.BlockSpec((D2, 16), lambda i: (0, 0)),
            pl.BlockSpec((1, 16), lambda i: (0, 0)),
        ],
        out_specs=[
            pl.BlockSpec((BN, W2C), lambda i: (i, 0)),
            pl.BlockSpec((BN, 16), lambda i: (i, 0)),
            pl.BlockSpec((1, 16), lambda i: (0, 0)),
            pl.BlockSpec((1, 16), lambda i: (0, 0)),
        ],
        out_shape=[
            jax.ShapeDtypeStruct((NN, W2C), jnp.float32),
            jax.ShapeDtypeStruct((NN, 16), jnp.float32),
            jax.ShapeDtypeStruct((1, 16), jnp.float32),
            jax.ShapeDtypeStruct((1, 16), jnp.float32),
        ],
    )(accp, hc1, ad1p, m1v, b1, pt, w2, aa2, ad2, pad2)


def _tc3_body(accp_r, hc2_r, ad_r, m2_r, b2_r, pt2_r, out_r):
    hc2 = hc2_r[...]
    h2 = hc2[:, 0:D2]
    a_s2 = hc2[:, D2:W2C]
    le = a_s2 + ad_r[...]
    le = jnp.maximum(le, 0.2 * le)
    lex = jnp.exp(le - m2_r[...])
    accp = accp_r[...]
    den16 = accp[0, :, D2:W2C] + accp[1, :, D2:W2C] + lex
    den = jnp.dot(den16, pt2_r[...], preferred_element_type=jnp.float32)
    esp = jnp.dot(lex, pt2_r[...], preferred_element_type=jnp.float32)
    acc = accp[0, :, 0:D2] + accp[1, :, 0:D2] + h2 * esp
    out2 = acc / (den + 1e-16) + b2_r[...]
    mx = jnp.max(out2, axis=1, keepdims=True)
    sh = out2 - mx
    out_r[...] = sh - jnp.log(jnp.sum(jnp.exp(sh), axis=1, keepdims=True))


def _tc3(accp, hc2, ad2p, m2v, b2, pt2):
    return pl.pallas_call(
        _tc3_body,
        grid=(NN // BN,),
        in_specs=[
            pl.BlockSpec((2, BN, W2C), lambda i: (0, i, 0)),
            pl.BlockSpec((BN, W2C), lambda i: (i, 0)),
            pl.BlockSpec((BN, 16), lambda i: (i, 0)),
            pl.BlockSpec((1, 16), lambda i: (0, 0)),
            pl.BlockSpec((1, D2), lambda i: (0, 0)),
            pl.BlockSpec((16, D2), lambda i: (0, 0)),
        ],
        out_specs=pl.BlockSpec((BN, D2), lambda i: (i, 0)),
        out_shape=jax.ShapeDtypeStruct((NN, D2), jnp.float32),
    )(accp, hc2, ad2p, m2v, b2, pt2)


def kernel(x, edge_index, W1, att_src1, att_dst1, b1, W2, att_src2,
           att_dst2, b2):
    lane = jnp.arange(16)
    hof = jax.nn.one_hot(jnp.arange(DIN) // CH, 16, dtype=jnp.float32)
    pa = hof * att_src1.reshape(DIN)[:, None]
    pd = hof * att_dst1.reshape(DIN)[:, None]
    pads = jnp.where(lane < NH, 0.0, NEG).reshape(1, 16).astype(jnp.float32)
    pt = hof.T

    hc1, ad1p, ms1, md1 = _tc1(x, W1, pa, pd, pads)

    m1 = ms1 + md1
    m1 = jnp.maximum(m1, 0.2 * m1)
    m1v = jnp.where(lane < NH, m1, 1e30).astype(jnp.float32)

    src2 = edge_index[0].reshape(EE // CHK, CHK).astype(jnp.int32)
    dst2 = edge_index[1].reshape(EE // CHK, CHK).astype(jnp.int32)

    sc1 = _make_sc_edge(DIN, CH, 100, 1, EE // (NW * 100))
    accp = sc1(src2, dst2, hc1, ad1p, m1v.reshape(16))

    aa2 = jnp.where(lane[None, :] == 0, att_src2.reshape(D2)[:, None], 0.0)
    ad2 = jnp.where(lane[None, :] == 0, att_dst2.reshape(D2)[:, None], 0.0)
    pad2 = jnp.where(lane == 0, 0.0, NEG).reshape(1, 16).astype(jnp.float32)

    hc2, ad2p, ms2, md2 = _tc2(
        accp, hc1, ad1p, m1v.reshape(1, 16), b1.reshape(1, DIN),
        pt, W2, aa2, ad2, pad2)

    m2 = ms2 + md2
    m2 = jnp.maximum(m2, 0.2 * m2)
    m2v = jnp.where(lane == 0, m2, 1e30).astype(jnp.float32)

    sc2 = _make_sc_edge(D2, D2, 200, 2, EE // (NW * 200))
    acc2p = sc2(src2, dst2, hc2, ad2p, m2v.reshape(16))

    pt2 = jnp.where(lane[:, None] == 0, 1.0, 0.0) * jnp.ones((16, D2))
    return _tc3(acc2p, hc2, ad2p, m2v.reshape(1, 16),
                b2.reshape(1, D2), pt2.astype(jnp.float32))
